# Initial kernel scaffold; baseline (speedup 1.0000x reference)
import jax
import jax.numpy as jnp
from jax.experimental import pallas as pl


def kernel(x, log_alpha, log_cp_cores):
    raise NotImplementedError("write your pallas kernel here")


# speedup vs baseline: 3.3753x; 3.3753x over previous
import math

import jax
import jax.numpy as jnp
from jax import lax
from jax.experimental import pallas as pl

DIM = 64
K = 64
C = 50
CP = 64
B = 1024
BETA = 1e-05
LOG_DIFF = math.log(BETA / C + 1e-15)
LOG_EQUAL = math.log(1 - (C - 1) / C * BETA + 1e-15)
A_CONST = math.exp(LOG_DIFF)
B_CONST = math.exp(LOG_EQUAL) - math.exp(LOG_DIFF)
NEG = -1e30


def _g_table_body(ct_ref, gt_ref):
    ct = ct_ref[...]
    e = jnp.exp(ct)
    s = jnp.sum(e, axis=1, keepdims=True)
    gt_ref[...] = jnp.log(A_CONST * s + B_CONST * e)


def _col(x_ref, d, n):
    ii = lax.broadcasted_iota(jnp.int32, (n, 1), 0)
    ed = jnp.where(ii == d, 1.0, 0.0).astype(jnp.float32)
    return lax.dot_general(x_ref[...], ed, (((1,), (0,)), ((), ())),
                           precision=lax.Precision.HIGHEST,
                           preferred_element_type=jnp.float32)


def _logz_body(x_ref, gt_ref, out_ref):
    d = pl.program_id(0)

    @pl.when(d == 0)
    def _():
        out_ref[...] = jnp.zeros_like(out_ref)

    xd = _col(x_ref, d, DIM).astype(jnp.int32)
    ii = lax.broadcasted_iota(jnp.int32, (B, CP), 1)
    onehot = jnp.where(xd == ii, 1.0, 0.0).astype(jnp.float32)
    gt = gt_ref[0]
    mm = lax.dot_general(onehot, gt, (((1,), (0,)), ((), ())),
                         precision=lax.Precision.HIGHEST,
                         preferred_element_type=jnp.float32)
    out_ref[...] += mm


def _kstar_body(logz_ref, alpha_ref, g1_ref, out_ref):
    lw = alpha_ref[...] + logz_ref[...]
    m = jnp.max(lw, axis=1, keepdims=True)
    s = jnp.log(jnp.sum(jnp.exp(lw - m), axis=1, keepdims=True)) + m
    v = g1_ref[...] + (lw - s)
    mv = jnp.max(v, axis=1, keepdims=True)
    ii = lax.broadcasted_iota(jnp.int32, (B, K), 1)
    out_ref[...] = jnp.min(jnp.where(v == mv, ii, K), axis=1, keepdims=True)


def _sample_body(ks_ref, x_ref, cores_ref, g2_ref, y_ref):
    d = pl.program_id(0)

    @pl.when(d == 0)
    def _():
        y_ref[...] = jnp.zeros_like(y_ref)

    ks = ks_ref[...]
    ii_k = lax.broadcasted_iota(jnp.int32, (B, K), 1)
    onehot = jnp.where(ks == ii_k, 1.0, 0.0).astype(jnp.float32)
    rows = lax.dot_general(onehot, cores_ref[0], (((1,), (0,)), ((), ())),
                           precision=lax.Precision.HIGHEST,
                           preferred_element_type=jnp.float32)
    xd = _col(x_ref, d, DIM).astype(jnp.int32)
    ii_c = lax.broadcasted_iota(jnp.int32, (B, CP), 1)
    pi = jnp.where(ii_c == xd, LOG_EQUAL, LOG_DIFF).astype(jnp.float32)
    v = (rows + pi) + g2_ref[0]
    v = jnp.where(ii_c < C, v, NEG)
    mv = jnp.max(v, axis=1, keepdims=True)
    yd = jnp.min(jnp.where(v == mv, ii_c, CP), axis=1, keepdims=True)
    lane_d = lax.broadcasted_iota(jnp.int32, (1, DIM), 1)
    y_ref[...] += yd * jnp.where(lane_d == d, 1, 0)


def kernel(x, log_alpha, log_cp_cores):
    f32 = jnp.float32
    skey = jax.random.key(42)
    g1 = jax.random.gumbel(jax.random.fold_in(skey, 0), (B, K), f32)
    g2 = jax.vmap(
        lambda i: jax.random.gumbel(jax.random.fold_in(skey, i), (B, C), f32)
    )(jnp.arange(1, DIM + 1))
    g2 = jnp.pad(g2, ((0, 0), (0, 0), (0, CP - C)))

    cores_pad = jnp.pad(log_cp_cores, ((0, 0), (0, 0), (0, CP - C)),
                        constant_values=NEG)
    cores_t = jnp.transpose(cores_pad, (0, 2, 1))
    x_f = x.astype(f32)

    gt = pl.pallas_call(
        _g_table_body,
        out_shape=jax.ShapeDtypeStruct((DIM, CP, K), f32),
    )(cores_t)

    logz = pl.pallas_call(
        _logz_body,
        grid=(DIM,),
        in_specs=[
            pl.BlockSpec((B, DIM), lambda d: (0, 0)),
            pl.BlockSpec((1, CP, K), lambda d: (d, 0, 0)),
        ],
        out_specs=pl.BlockSpec((B, K), lambda d: (0, 0)),
        out_shape=jax.ShapeDtypeStruct((B, K), f32),
    )(x_f, gt)

    k_star = pl.pallas_call(
        _kstar_body,
        out_shape=jax.ShapeDtypeStruct((B, 1), jnp.int32),
    )(logz, jnp.reshape(log_alpha, (1, K)), g1)

    y = pl.pallas_call(
        _sample_body,
        grid=(DIM,),
        in_specs=[
            pl.BlockSpec((B, 1), lambda d: (0, 0)),
            pl.BlockSpec((B, DIM), lambda d: (0, 0)),
            pl.BlockSpec((1, K, CP), lambda d: (d, 0, 0)),
            pl.BlockSpec((1, B, CP), lambda d: (d, 0, 0)),
        ],
        out_specs=pl.BlockSpec((B, DIM), lambda d: (0, 0)),
        out_shape=jax.ShapeDtypeStruct((B, DIM), jnp.int32),
    )(k_star, x_f, cores_pad, g2)

    return y


# speedup vs baseline: 4.8698x; 1.4428x over previous
import functools
import math

import jax
import jax.numpy as jnp
from jax import lax
from jax.experimental import pallas as pl
from jax.experimental.pallas import tpu as pltpu
from jax.experimental.pallas import tpu_sc as plsc

DIM = 64
K = 64
C = 50
CP = 64
B = 1024
BETA = 1e-05
LOG_DIFF = math.log(BETA / C + 1e-15)
LOG_EQUAL = math.log(1 - (C - 1) / C * BETA + 1e-15)
A_CONST = math.exp(LOG_DIFF)
B_CONST = math.exp(LOG_EQUAL) - math.exp(LOG_DIFF)
NEG = -1e30

NW = 32
SPW = B // NW
L = 16


def _g_table_body(ct_ref, gt_ref):
    ct = ct_ref[...]
    e = jnp.exp(ct)
    s = jnp.sum(e, axis=1, keepdims=True)
    gt_ref[...] = jnp.log(A_CONST * s + B_CONST * e)


def _logz_sc_body(x_hbm, gt_hbm, out_hbm, xf, off, idx, rows0, rows1, res,
                  sem0, sem1):
    wid = lax.axis_index("s") * 2 + lax.axis_index("c")
    base = wid * (SPW * DIM)

    pltpu.sync_copy(x_hbm.at[pl.ds(base, SPW * DIM)], xf)
    for q in range(DIM // L):
        off[pl.ds(L * q, L)] = lax.iota(jnp.int32, L) * CP + (L * CP) * q
    for t in range(SPW * DIM // L):
        q = t % (DIM // L)
        idx[pl.ds(L * t, L)] = xf[pl.ds(L * t, L)] + off[pl.ds(L * q, L)]

    def fire(s, rbuf, sem):
        pltpu.async_copy(gt_hbm.at[idx.at[pl.ds(s * DIM, DIM)]], rbuf, sem)

    def wait(rbuf, sem):
        pltpu.make_async_copy(gt_hbm.at[idx.at[pl.ds(0, DIM)]], rbuf, sem).wait()

    def acc_sample(s, rbuf):
        regs = [rbuf[0, pl.ds(L * q, L)] for q in range(K // L)]
        for j in range(1, DIM):
            for q in range(K // L):
                regs[q] = regs[q] + rbuf[j, pl.ds(L * q, L)]
        for q in range(K // L):
            res[pl.ds(s * K + L * q, L)] = regs[q]

    fire(0, rows0, sem0)
    fire(1, rows1, sem1)

    def body(i, carry):
        s0 = 2 * i
        wait(rows0, sem0)
        acc_sample(s0, rows0)

        @pl.when(s0 + 2 < SPW)
        def _():
            fire(s0 + 2, rows0, sem0)

        wait(rows1, sem1)
        acc_sample(s0 + 1, rows1)

        @pl.when(s0 + 3 < SPW)
        def _():
            fire(s0 + 3, rows1, sem1)

        return carry

    lax.fori_loop(0, SPW // 2, body, 0)
    pltpu.sync_copy(res, out_hbm.at[pl.ds(wid * SPW * K, SPW * K)])


def _logz_sc(x_flat, gt_flat):
    f32 = jnp.float32
    return pl.kernel(
        _logz_sc_body,
        out_type=jax.ShapeDtypeStruct((B * K,), f32),
        mesh=plsc.VectorSubcoreMesh(core_axis_name="c", subcore_axis_name="s"),
        compiler_params=pltpu.CompilerParams(use_tc_tiling_on_sc=False),
        scratch_types=[
            pltpu.VMEM((SPW * DIM,), jnp.int32),
            pltpu.VMEM((DIM,), jnp.int32),
            pltpu.VMEM((SPW * DIM,), jnp.int32),
            pltpu.VMEM((DIM, K), f32),
            pltpu.VMEM((DIM, K), f32),
            pltpu.VMEM((SPW * K,), f32),
            pltpu.SemaphoreType.DMA,
            pltpu.SemaphoreType.DMA,
        ],
    )(x_flat, gt_flat)


def _kstar_body(logz_ref, alpha_ref, g1_ref, out_ref):
    lw = alpha_ref[...] + logz_ref[...]
    m = jnp.max(lw, axis=1, keepdims=True)
    s = jnp.log(jnp.sum(jnp.exp(lw - m), axis=1, keepdims=True)) + m
    v = g1_ref[...] + (lw - s)
    mv = jnp.max(v, axis=1, keepdims=True)
    ii = lax.broadcasted_iota(jnp.int32, (B, K), 1)
    out_ref[...] = jnp.min(jnp.where(v == mv, ii, K), axis=1, keepdims=True)


NCHUNK = 128 // SPW
NT = DIM // NCHUNK


def _rows_sc_body(ks_hbm, cores_hbm, out_hbm, kv, idx, buf0, buf1,
                  gsem0, gsem1, wsem0, wsem1):
    wid = lax.axis_index("s") * 2 + lax.axis_index("c")
    base = wid * SPW

    pltpu.sync_copy(ks_hbm.at[pl.ds(base, SPW)], kv)
    for d in range(DIM):
        for q in range(SPW // L):
            idx[pl.ds(d * SPW + L * q, L)] = kv[pl.ds(L * q, L)] + d * K

    bufs = (buf0, buf1)
    gsems = (gsem0, gsem1)
    wsems = (wsem0, wsem1)

    def fire_gather(t):
        pltpu.async_copy(
            cores_hbm.at[idx.at[pl.ds(t * NCHUNK * SPW, NCHUNK * SPW)]],
            bufs[t % 2], gsems[t % 2])

    def drain_writes(t):
        pltpu.make_async_copy(bufs[t % 2],
                              out_hbm.at[pl.ds(0, NCHUNK * SPW)],
                              wsems[t % 2]).wait()

    def fire_writes(t):
        for j in range(NCHUNK):
            d = t * NCHUNK + j
            pltpu.async_copy(bufs[t % 2].at[pl.ds(j * SPW, SPW)],
                             out_hbm.at[pl.ds(d * B + base, SPW)],
                             wsems[t % 2])

    fire_gather(0)
    for t in range(NT):
        if t + 1 < NT:
            if t >= 1:
                drain_writes(t + 1)
            fire_gather(t + 1)
        pltpu.make_async_copy(
            cores_hbm.at[idx.at[pl.ds(0, NCHUNK * SPW)]],
            bufs[t % 2], gsems[t % 2]).wait()
        fire_writes(t)
    drain_writes(NT - 2)
    drain_writes(NT - 1)


def _rows_sc(ks_flat, cores_rows):
    f32 = jnp.float32
    return pl.kernel(
        _rows_sc_body,
        out_type=jax.ShapeDtypeStruct((DIM * B, CP), f32),
        mesh=plsc.VectorSubcoreMesh(core_axis_name="c", subcore_axis_name="s"),
        compiler_params=pltpu.CompilerParams(use_tc_tiling_on_sc=False),
        scratch_types=[
            pltpu.VMEM((SPW,), jnp.int32),
            pltpu.VMEM((DIM * SPW,), jnp.int32),
            pltpu.VMEM((NCHUNK * SPW, CP), f32),
            pltpu.VMEM((NCHUNK * SPW, CP), f32),
            pltpu.SemaphoreType.DMA,
            pltpu.SemaphoreType.DMA,
            pltpu.SemaphoreType.DMA,
            pltpu.SemaphoreType.DMA,
        ],
    )(ks_flat, cores_rows)


DCHUNK = 8


def _sample_body(r_ref, xt_ref, g2_ref, y_ref):
    ii_c = lax.broadcasted_iota(jnp.int32, (DCHUNK, B, CP), 2)
    pi = jnp.where(ii_c == xt_ref[...], LOG_EQUAL, LOG_DIFF).astype(jnp.float32)
    v = (r_ref[...] + pi) + g2_ref[...]
    v = jnp.where(ii_c < C, v, NEG)
    mv = jnp.max(v, axis=2, keepdims=True)
    y_ref[...] = jnp.min(jnp.where(v == mv, ii_c, CP), axis=2)


def kernel(x, log_alpha, log_cp_cores):
    f32 = jnp.float32
    skey = jax.random.key(42)
    g1 = jax.random.gumbel(jax.random.fold_in(skey, 0), (B, K), f32)
    g2 = jax.vmap(
        lambda i: jax.random.gumbel(jax.random.fold_in(skey, i), (B, C), f32)
    )(jnp.arange(1, DIM + 1))
    g2 = jnp.pad(g2, ((0, 0), (0, 0), (0, CP - C)))

    cores_pad = jnp.pad(log_cp_cores, ((0, 0), (0, 0), (0, CP - C)),
                        constant_values=NEG)
    cores_t = jnp.transpose(cores_pad, (0, 2, 1))

    gt = pl.pallas_call(
        _g_table_body,
        out_shape=jax.ShapeDtypeStruct((DIM, CP, K), f32),
    )(cores_t)

    logz = _logz_sc(x.reshape(-1), gt.reshape(DIM * CP, K)).reshape(B, K)

    k_star = pl.pallas_call(
        _kstar_body,
        out_shape=jax.ShapeDtypeStruct((B, 1), jnp.int32),
    )(logz, jnp.reshape(log_alpha, (1, K)), g1)

    rows = _rows_sc(k_star.reshape(B), cores_pad.reshape(DIM * K, CP))
    rows = rows.reshape(DIM, B, CP)

    xt3 = jnp.transpose(x, (1, 0)).reshape(DIM, B, 1)

    yt = pl.pallas_call(
        _sample_body,
        grid=(DIM // DCHUNK,),
        in_specs=[
            pl.BlockSpec((DCHUNK, B, CP), lambda t: (t, 0, 0)),
            pl.BlockSpec((DCHUNK, B, 1), lambda t: (t, 0, 0)),
            pl.BlockSpec((DCHUNK, B, CP), lambda t: (t, 0, 0)),
        ],
        out_specs=pl.BlockSpec((DCHUNK, B), lambda t: (t, 0)),
        out_shape=jax.ShapeDtypeStruct((DIM, B), jnp.int32),
    )(rows, xt3, g2)

    return jnp.transpose(yt, (1, 0))
